# baseline (speedup 1.0000x reference)
import functools

import jax
import jax.numpy as jnp
from jax import lax
from jax.experimental import pallas as pl
from jax.experimental.pallas import tpu as pltpu
from jax.experimental.pallas import tpu_sc as plsc

N_USERS = 1000000
N_ITEMS = 1000000
N_F = 64
BATCH = 16384

NC = 2
NS = 16
L = 16
NW = NC * NS
B_PER_W = BATCH // NW
GROUPS = B_PER_W // L


def _sc_body(user_hbm, item_hbm, gm_hbm, wpu_hbm, wpi_hbm, bu_hbm, bi_hbm,
             out_hbm,
             uidx_v, iidx_v, urows_v, irows_v, bu_v, bi_v, gm_v, out_v, sem):
    wid = lax.axis_index("s") * NC + lax.axis_index("c")
    base = wid * B_PER_W

    pltpu.sync_copy(user_hbm.at[pl.ds(base, B_PER_W)], uidx_v)
    pltpu.sync_copy(item_hbm.at[pl.ds(base, B_PER_W)], iidx_v)
    pltpu.sync_copy(gm_hbm, gm_v)

    c1 = pltpu.async_copy(wpu_hbm.at[uidx_v], urows_v, sem)
    c2 = pltpu.async_copy(wpi_hbm.at[iidx_v], irows_v, sem)
    c3 = pltpu.async_copy(bu_hbm.at[uidx_v], bu_v, sem)
    c4 = pltpu.async_copy(bi_hbm.at[iidx_v], bi_v, sem)
    c1.wait()
    c2.wait()
    c3.wait()
    c4.wait()

    gm = gm_v[...]
    lanes = lax.iota(jnp.int32, L)

    def group(g, carry):
        row0 = g * L
        rows = row0 + lanes
        acc = jnp.zeros((L,), jnp.float32)
        for j in range(N_F):
            cols = jnp.full((L,), j, jnp.int32)
            u = plsc.load_gather(urows_v, [rows, cols])
            it = plsc.load_gather(irows_v, [rows, cols])
            acc = acc + u * it
        out_v[pl.ds(row0, L)] = (acc + bu_v[pl.ds(row0, L)]
                                 + bi_v[pl.ds(row0, L)] + gm)
        return carry

    lax.fori_loop(0, GROUPS, group, 0)
    pltpu.sync_copy(out_v, out_hbm.at[pl.ds(base, B_PER_W)])


@jax.jit
def _sc_call(user, item, gm16, WPU, WPI, BU, BI):
    mesh = plsc.VectorSubcoreMesh(core_axis_name="c", subcore_axis_name="s")
    f = pl.kernel(
        _sc_body,
        out_type=jax.ShapeDtypeStruct((BATCH,), jnp.float32),
        mesh=mesh,
        scratch_types=[
            pltpu.VMEM((B_PER_W,), jnp.int32),
            pltpu.VMEM((B_PER_W,), jnp.int32),
            pltpu.VMEM((B_PER_W, N_F), jnp.float32),
            pltpu.VMEM((B_PER_W, N_F), jnp.float32),
            pltpu.VMEM((B_PER_W,), jnp.float32),
            pltpu.VMEM((B_PER_W,), jnp.float32),
            pltpu.VMEM((L,), jnp.float32),
            pltpu.VMEM((B_PER_W,), jnp.float32),
            pltpu.SemaphoreType.DMA,
        ],
        compiler_params=pltpu.CompilerParams(
            needs_layout_passes=False, use_tc_tiling_on_sc=False),
    )
    return f(user, item, gm16, WPU, WPI, BU, BI)


def kernel(user, item, global_mean, WPU, WPI, BU, BI):
    user = user.astype(jnp.int32)
    item = item.astype(jnp.int32)
    gm16 = jnp.broadcast_to(global_mean.astype(jnp.float32), (L,))
    return _sc_call(user, item, gm16, WPU, WPI, BU, BI)
